# Initial kernel scaffold; baseline (speedup 1.0000x reference)
import jax
import jax.numpy as jnp
from jax.experimental import pallas as pl


def kernel(x, embed_table, W1, b1, W2, b2):
    raise NotImplementedError("write your pallas kernel here")


# speedup vs baseline: 2.5535x; 2.5535x over previous
import functools

import jax
import jax.numpy as jnp
from jax import lax
from jax.experimental import pallas as pl
from jax.experimental.pallas import tpu as pltpu
from jax.experimental.pallas import tpu_sc as plsc

B = 4096
SEQ = 50
EMB = 128
HID = 2048
CLS = 1000

NC = 2
NS = 16
NW = NC * NS
TOT = B * SEQ
ROWS_PER_W = TOT // NW
GROUP = 128
NGROUPS = ROWS_PER_W // GROUP


def _gather_body(idx_hbm, table_hbm, out_hbm, idx_v, rows_v, sems):
    wid = lax.axis_index("s") * NC + lax.axis_index("c")
    base = wid * ROWS_PER_W
    pltpu.sync_copy(idx_hbm.at[wid], idx_v)

    cp0 = pltpu.make_async_copy(table_hbm.at[idx_v.at[0]], rows_v.at[0],
                                sems.at[0])
    cp0.start()

    def body(j, _):
        nxt = (j + 1) % 2
        cur = j % 2

        @pl.when(j + 1 < NGROUPS)
        def _():
            cp = pltpu.make_async_copy(table_hbm.at[idx_v.at[j + 1]],
                                       rows_v.at[nxt], sems.at[nxt])
            cp.start()

        pltpu.make_async_copy(table_hbm.at[idx_v.at[j]], rows_v.at[cur],
                              sems.at[cur]).wait()
        pltpu.sync_copy(rows_v.at[cur],
                        out_hbm.at[pl.ds(base + j * GROUP, GROUP)])
        return 0

    lax.fori_loop(0, NGROUPS, body, 0, unroll=2)


def _sc_gather(xf, table):
    mesh = plsc.VectorSubcoreMesh(core_axis_name="c", subcore_axis_name="s")
    return pl.kernel(
        _gather_body,
        mesh=mesh,
        out_type=jax.ShapeDtypeStruct((TOT, EMB), jnp.float32),
        scratch_types=[
            pltpu.VMEM((NGROUPS, GROUP), jnp.int32),
            pltpu.VMEM((2, GROUP, EMB), jnp.float32),
            pltpu.SemaphoreType.DMA((2,)),
        ],
    )(xf, table)


BM = 512
BK = 1280
NK = (SEQ * EMB) // BK


def _mlp_body(flat_ref, w1_ref, b1_ref, w2_ref, b2_ref, out_ref, acc_ref):
    k = pl.program_id(1)

    @pl.when(k == 0)
    def _():
        acc_ref[...] = jnp.broadcast_to(b1_ref[...], acc_ref.shape)

    acc_ref[...] += jnp.dot(flat_ref[...], w1_ref[...],
                            preferred_element_type=jnp.float32)

    @pl.when(k == NK - 1)
    def _():
        out_ref[...] = (
            jnp.dot(acc_ref[...], w2_ref[...],
                    preferred_element_type=jnp.float32)
            + b2_ref[...]
        )


def _tc_mlp(flat, W1, b1, W2, b2):
    return pl.pallas_call(
        _mlp_body,
        grid=(B // BM, NK),
        in_specs=[
            pl.BlockSpec((BM, BK), lambda m, k: (m, k)),
            pl.BlockSpec((BK, HID), lambda m, k: (k, 0)),
            pl.BlockSpec((1, HID), lambda m, k: (0, 0)),
            pl.BlockSpec((HID, CLS), lambda m, k: (0, 0)),
            pl.BlockSpec((1, CLS), lambda m, k: (0, 0)),
        ],
        out_specs=pl.BlockSpec((BM, CLS), lambda m, k: (m, 0)),
        out_shape=jax.ShapeDtypeStruct((B, CLS), jnp.float32),
        scratch_shapes=[pltpu.VMEM((BM, HID), jnp.float32)],
        compiler_params=pltpu.CompilerParams(
            dimension_semantics=("parallel", "arbitrary"),
        ),
    )(flat, W1, b1.reshape(1, HID), W2, b2.reshape(1, CLS))


def kernel(x, embed_table, W1, b1, W2, b2):
    xf = x.astype(jnp.int32).reshape(NW, NGROUPS, GROUP)
    flat = _sc_gather(xf, embed_table)
    flat = flat.reshape(B, SEQ * EMB)
    return _tc_mlp(flat, W1, b1, W2, b2)


# speedup vs baseline: 2.7301x; 1.0692x over previous
import functools

import jax
import jax.numpy as jnp
from jax import lax
from jax.experimental import pallas as pl
from jax.experimental.pallas import tpu as pltpu
from jax.experimental.pallas import tpu_sc as plsc

B = 4096
SEQ = 50
EMB = 128
HID = 2048
CLS = 1000

EMB2 = EMB // 2

NC = 2
NS = 16
NW = NC * NS
TOT = B * SEQ
ROWS_PER_W = TOT // NW
GROUP = 128
NGROUPS = ROWS_PER_W // GROUP


def _gather_body(idx_hbm, table_hbm, out_hbm, idx_v, rows_v, sems):
    wid = lax.axis_index("s") * NC + lax.axis_index("c")
    base = wid * ROWS_PER_W
    pltpu.sync_copy(idx_hbm.at[wid], idx_v)

    cp0 = pltpu.make_async_copy(table_hbm.at[idx_v.at[0]], rows_v.at[0],
                                sems.at[0])
    cp0.start()

    def body(j, _):
        nxt = (j + 1) % 2
        cur = j % 2

        @pl.when(j + 1 < NGROUPS)
        def _():
            cp = pltpu.make_async_copy(table_hbm.at[idx_v.at[j + 1]],
                                       rows_v.at[nxt], sems.at[nxt])
            cp.start()

        pltpu.make_async_copy(table_hbm.at[idx_v.at[j]], rows_v.at[cur],
                              sems.at[cur]).wait()
        pltpu.sync_copy(rows_v.at[cur],
                        out_hbm.at[pl.ds(base + j * GROUP, GROUP)])
        return 0

    lax.fori_loop(0, NGROUPS, body, 0, unroll=2)


def _sc_gather(xf, table):
    mesh = plsc.VectorSubcoreMesh(core_axis_name="c", subcore_axis_name="s")
    return pl.kernel(
        _gather_body,
        mesh=mesh,
        out_type=jax.ShapeDtypeStruct((TOT, EMB), jnp.float32),
        scratch_types=[
            pltpu.VMEM((NGROUPS, GROUP), jnp.int32),
            pltpu.VMEM((2, GROUP, EMB), jnp.float32),
            pltpu.SemaphoreType.DMA((2,)),
        ],
    )(xf, table)


BM = 1024
BK = 1280
NK = (SEQ * EMB) // BK


def _mlp_body(flat_ref, w1_ref, b1_ref, w2_ref, b2_ref, out_ref, acc_ref):
    k = pl.program_id(1)

    @pl.when(k == 0)
    def _():
        acc_ref[...] = jnp.broadcast_to(b1_ref[...], acc_ref.shape)

    acc_ref[...] += jnp.dot(flat_ref[...].astype(jnp.bfloat16), w1_ref[...],
                            preferred_element_type=jnp.float32)

    @pl.when(k == NK - 1)
    def _():
        out_ref[...] = (
            jnp.dot(acc_ref[...].astype(jnp.bfloat16), w2_ref[...],
                    preferred_element_type=jnp.float32)
            + b2_ref[...]
        )


def _tc_mlp(flat, W1, b1, W2, b2):
    return pl.pallas_call(
        _mlp_body,
        grid=(B // BM, NK),
        in_specs=[
            pl.BlockSpec((BM, BK), lambda m, k: (m, k)),
            pl.BlockSpec((BK, HID), lambda m, k: (k, 0)),
            pl.BlockSpec((1, HID), lambda m, k: (0, 0)),
            pl.BlockSpec((HID, CLS), lambda m, k: (0, 0)),
            pl.BlockSpec((1, CLS), lambda m, k: (0, 0)),
        ],
        out_specs=pl.BlockSpec((BM, CLS), lambda m, k: (m, 0)),
        out_shape=jax.ShapeDtypeStruct((B, CLS), jnp.float32),
        scratch_shapes=[pltpu.VMEM((BM, HID), jnp.float32)],
        compiler_params=pltpu.CompilerParams(
            dimension_semantics=("parallel", "arbitrary"),
        ),
    )(flat, W1, b1.reshape(1, HID), W2, b2.reshape(1, CLS))


def kernel(x, embed_table, W1, b1, W2, b2):
    xf = x.astype(jnp.int32).reshape(NW, NGROUPS, GROUP)
    flat = _sc_gather(xf, embed_table)
    flat = flat.reshape(B, SEQ * EMB)
    return _tc_mlp(flat, W1.astype(jnp.bfloat16), b1,
                   W2.astype(jnp.bfloat16), b2)


# speedup vs baseline: 2.7408x; 1.0039x over previous
import functools

import jax
import jax.numpy as jnp
from jax import lax
from jax.experimental import pallas as pl
from jax.experimental.pallas import tpu as pltpu
from jax.experimental.pallas import tpu_sc as plsc

B = 4096
SEQ = 50
EMB = 128
HID = 2048
CLS = 1000

EMB2 = EMB // 2

NC = 2
NS = 16
NW = NC * NS
TOT = B * SEQ
ROWS_PER_W = TOT // NW
GROUP = 128
NGROUPS = ROWS_PER_W // GROUP


def _gather_body(idx_hbm, table_hbm, out_hbm, idx_v, rows_v, sems):
    wid = lax.axis_index("s") * NC + lax.axis_index("c")
    base = wid * ROWS_PER_W
    pltpu.sync_copy(idx_hbm.at[wid], idx_v)

    cp0 = pltpu.make_async_copy(table_hbm.at[idx_v.at[0]], rows_v.at[0],
                                sems.at[0])
    cp0.start()

    def body(j, _):
        nxt = (j + 1) % 2
        cur = j % 2

        @pl.when(j + 1 < NGROUPS)
        def _():
            cp = pltpu.make_async_copy(table_hbm.at[idx_v.at[j + 1]],
                                       rows_v.at[nxt], sems.at[nxt])
            cp.start()

        pltpu.make_async_copy(table_hbm.at[idx_v.at[j]], rows_v.at[cur],
                              sems.at[cur]).wait()
        pltpu.sync_copy(rows_v.at[cur],
                        out_hbm.at[pl.ds(base + j * GROUP, GROUP)])
        return 0

    lax.fori_loop(0, NGROUPS, body, 0, unroll=2)


def _sc_gather(xf, table):
    mesh = plsc.VectorSubcoreMesh(core_axis_name="c", subcore_axis_name="s")
    return pl.kernel(
        _gather_body,
        mesh=mesh,
        out_type=jax.ShapeDtypeStruct((TOT, EMB), jnp.float32),
        scratch_types=[
            pltpu.VMEM((NGROUPS, GROUP), jnp.int32),
            pltpu.VMEM((2, GROUP, EMB), jnp.float32),
            pltpu.SemaphoreType.DMA((2,)),
        ],
    )(xf, table)


BM = 256
KTOT = SEQ * EMB


def _mlp_body(flat_ref, w1_ref, b1_ref, w2_ref, b2_ref, out_ref):
    h = jnp.dot(flat_ref[...].astype(jnp.bfloat16), w1_ref[...],
                preferred_element_type=jnp.float32) + b1_ref[...]
    out_ref[...] = (
        jnp.dot(h.astype(jnp.bfloat16), w2_ref[...],
                preferred_element_type=jnp.float32)
        + b2_ref[...]
    )


def _tc_mlp(flat, W1, b1, W2, b2):
    return pl.pallas_call(
        _mlp_body,
        grid=(B // BM,),
        in_specs=[
            pl.BlockSpec((BM, KTOT), lambda m: (m, 0)),
            pl.BlockSpec((KTOT, HID), lambda m: (0, 0)),
            pl.BlockSpec((1, HID), lambda m: (0, 0)),
            pl.BlockSpec((HID, CLS), lambda m: (0, 0)),
            pl.BlockSpec((1, CLS), lambda m: (0, 0)),
        ],
        out_specs=pl.BlockSpec((BM, CLS), lambda m: (m, 0)),
        out_shape=jax.ShapeDtypeStruct((B, CLS), jnp.float32),
        compiler_params=pltpu.CompilerParams(
            dimension_semantics=("arbitrary",),
        ),
    )(flat, W1, b1.reshape(1, HID), W2, b2.reshape(1, CLS))


def kernel(x, embed_table, W1, b1, W2, b2):
    xf = x.astype(jnp.int32).reshape(NW, NGROUPS, GROUP)
    flat = _sc_gather(xf, embed_table)
    flat = flat.reshape(B, SEQ * EMB)
    return _tc_mlp(flat, W1.astype(jnp.bfloat16), b1,
                   W2.astype(jnp.bfloat16), b2)


# speedup vs baseline: 3.7127x; 1.3546x over previous
import functools

import jax
import jax.numpy as jnp
from jax import lax
from jax.experimental import pallas as pl
from jax.experimental.pallas import tpu as pltpu
from jax.experimental.pallas import tpu_sc as plsc

B = 4096
SEQ = 50
EMB = 128
HID = 2048
CLS = 1000

EMB2 = EMB // 2

NC = 2
NS = 16
NW = NC * NS
TOT = B * SEQ
ROWS_PER_W = TOT // NW
GROUP = 128
NGROUPS = ROWS_PER_W // GROUP


def _gather_body(idx_hbm, table_hbm, out_hbm, idx_v, rows_v, sems):
    wid = lax.axis_index("s") * NC + lax.axis_index("c")
    row0 = wid * GROUP
    pltpu.sync_copy(idx_hbm.at[wid], idx_v)

    cp0 = pltpu.make_async_copy(table_hbm.at[idx_v.at[0]], rows_v.at[0],
                                sems.at[0])
    cp0.start()

    def body(j, _):
        nxt = (j + 1) % 2
        cur = j % 2

        @pl.when(j + 1 < NGROUPS)
        def _():
            cp = pltpu.make_async_copy(table_hbm.at[idx_v.at[j + 1]],
                                       rows_v.at[nxt], sems.at[nxt])
            cp.start()

        pltpu.make_async_copy(table_hbm.at[idx_v.at[j]], rows_v.at[cur],
                              sems.at[cur]).wait()
        pltpu.sync_copy(rows_v.at[cur],
                        out_hbm.at[pl.ds(row0, GROUP), pl.ds(j * EMB, EMB)])
        return 0

    lax.fori_loop(0, NGROUPS, body, 0, unroll=2)


def _sc_gather(xf, table):
    mesh = plsc.VectorSubcoreMesh(core_axis_name="c", subcore_axis_name="s")
    return pl.kernel(
        _gather_body,
        mesh=mesh,
        out_type=jax.ShapeDtypeStruct((B, SEQ * EMB), jnp.float32),
        scratch_types=[
            pltpu.VMEM((NGROUPS, GROUP), jnp.int32),
            pltpu.VMEM((2, GROUP, EMB), jnp.float32),
            pltpu.SemaphoreType.DMA((2,)),
        ],
    )(xf, table)


BM = 256
KTOT = SEQ * EMB


def _mlp_body(flat_ref, w1_ref, b1_ref, w2_ref, b2_ref, out_ref):
    h = jnp.dot(flat_ref[...].astype(jnp.bfloat16), w1_ref[...],
                preferred_element_type=jnp.float32) + b1_ref[...]
    out_ref[...] = (
        jnp.dot(h.astype(jnp.bfloat16), w2_ref[...],
                preferred_element_type=jnp.float32)
        + b2_ref[...]
    )


def _tc_mlp(flat, W1, b1, W2, b2):
    return pl.pallas_call(
        _mlp_body,
        grid=(B // BM,),
        in_specs=[
            pl.BlockSpec((BM, KTOT), lambda m: (m, 0)),
            pl.BlockSpec((KTOT, HID), lambda m: (0, 0)),
            pl.BlockSpec((1, HID), lambda m: (0, 0)),
            pl.BlockSpec((HID, CLS), lambda m: (0, 0)),
            pl.BlockSpec((1, CLS), lambda m: (0, 0)),
        ],
        out_specs=pl.BlockSpec((BM, CLS), lambda m: (m, 0)),
        out_shape=jax.ShapeDtypeStruct((B, CLS), jnp.float32),
        compiler_params=pltpu.CompilerParams(
            dimension_semantics=("arbitrary",),
        ),
    )(flat, W1, b1.reshape(1, HID), W2, b2.reshape(1, CLS))


def kernel(x, embed_table, W1, b1, W2, b2):
    xt = x.astype(jnp.int32).reshape(NW, GROUP, SEQ).transpose(0, 2, 1)
    flat = _sc_gather(xt, embed_table)
    return _tc_mlp(flat, W1.astype(jnp.bfloat16), b1,
                   W2.astype(jnp.bfloat16), b2)


# speedup vs baseline: 3.7297x; 1.0046x over previous
import functools

import jax
import jax.numpy as jnp
from jax import lax
from jax.experimental import pallas as pl
from jax.experimental.pallas import tpu as pltpu
from jax.experimental.pallas import tpu_sc as plsc

B = 4096
SEQ = 50
EMB = 128
HID = 2048
CLS = 1000

EMB2 = EMB // 2

NC = 2
NS = 16
NW = NC * NS
TOT = B * SEQ
ROWS_PER_W = TOT // NW
GROUP = 128
NGROUPS = ROWS_PER_W // GROUP
NBUF = 4


def _gather_body(idx_hbm, table_hbm, out_hbm, idx_v, rows_v, gsems, wsems):
    wid = lax.axis_index("s") * NC + lax.axis_index("c")
    row0 = wid * GROUP
    pltpu.sync_copy(idx_hbm.at[wid], idx_v)

    def g(j):
        return pltpu.make_async_copy(table_hbm.at[idx_v.at[j]],
                                     rows_v.at[j % NBUF], gsems.at[j % NBUF])

    def w(j):
        return pltpu.make_async_copy(
            rows_v.at[j % NBUF],
            out_hbm.at[pl.ds(row0, GROUP), pl.ds(j * EMB, EMB)],
            wsems.at[j % NBUF])

    for j0 in range(NBUF - 1):
        g(j0).start()

    def body(j, _):
        g(j).wait()
        w(j).start()
        nj = j + NBUF - 1

        @pl.when(nj < NGROUPS)
        def _():
            @pl.when(j >= 1)
            def _():
                w(j - 1).wait()

            g(nj).start()

        return 0

    lax.fori_loop(0, NGROUPS, body, 0)
    for t in range(NBUF):
        w(NGROUPS - NBUF + t).wait()


def _sc_gather(xf, table):
    mesh = plsc.VectorSubcoreMesh(core_axis_name="c", subcore_axis_name="s")
    return pl.kernel(
        _gather_body,
        mesh=mesh,
        out_type=jax.ShapeDtypeStruct((B, SEQ * EMB), jnp.float32),
        scratch_types=[
            pltpu.VMEM((NGROUPS, GROUP), jnp.int32),
            pltpu.VMEM((NBUF, GROUP, EMB), jnp.float32),
            pltpu.SemaphoreType.DMA((NBUF,)),
            pltpu.SemaphoreType.DMA((NBUF,)),
        ],
    )(xf, table)


BM = 256
KTOT = SEQ * EMB


def _mlp_body(flat_ref, w1_ref, b1_ref, w2_ref, b2_ref, out_ref):
    h = jnp.dot(flat_ref[...].astype(jnp.bfloat16), w1_ref[...],
                preferred_element_type=jnp.float32) + b1_ref[...]
    out_ref[...] = (
        jnp.dot(h.astype(jnp.bfloat16), w2_ref[...],
                preferred_element_type=jnp.float32)
        + b2_ref[...]
    )


def _tc_mlp(flat, W1, b1, W2, b2):
    return pl.pallas_call(
        _mlp_body,
        grid=(B // BM,),
        in_specs=[
            pl.BlockSpec((BM, KTOT), lambda m: (m, 0)),
            pl.BlockSpec((KTOT, HID), lambda m: (0, 0)),
            pl.BlockSpec((1, HID), lambda m: (0, 0)),
            pl.BlockSpec((HID, CLS), lambda m: (0, 0)),
            pl.BlockSpec((1, CLS), lambda m: (0, 0)),
        ],
        out_specs=pl.BlockSpec((BM, CLS), lambda m: (m, 0)),
        out_shape=jax.ShapeDtypeStruct((B, CLS), jnp.float32),
        compiler_params=pltpu.CompilerParams(
            dimension_semantics=("arbitrary",),
        ),
    )(flat, W1, b1.reshape(1, HID), W2, b2.reshape(1, CLS))


def kernel(x, embed_table, W1, b1, W2, b2):
    xt = x.astype(jnp.int32).reshape(NW, GROUP, SEQ).transpose(0, 2, 1)
    flat = _sc_gather(xt, embed_table)
    return _tc_mlp(flat, W1.astype(jnp.bfloat16), b1,
                   W2.astype(jnp.bfloat16), b2)
